# baseline (speedup 1.0000x reference)
import functools

import jax
import jax.numpy as jnp
from jax import lax
from jax.experimental import pallas as pl
from jax.experimental.pallas import tpu as pltpu
from jax.experimental.pallas import tpu_sc as plsc

NC = 2
NS = 16
NW = NC * NS
CW = 128
NB = 10
K = 5


@functools.cache
def _build(n_rows: int, vocab: int, dim: int):
    assert n_rows % (NW * CW) == 0
    ch = n_rows // (NW * CW)
    assert ch % NB == 0 and ch >= 2 * NB
    n_grp = ch // NB

    def body(table_hbm, idx_hbm, out_hbm, idx_v, *rest):
        bufs = rest[:NB]
        gsems = rest[NB:2 * NB]
        osems = rest[2 * NB:]
        wid = lax.axis_index("s") * NC + lax.axis_index("c")
        row0 = wid * ch * CW

        def gather(jj, slot):
            pltpu.async_copy(
                table_hbm.at[idx_v.at[jj]], bufs[slot], gsems[slot])

        def gather_wait(slot):
            pltpu.make_async_copy(
                table_hbm.at[idx_v.at[slot]], bufs[slot], gsems[slot]).wait()

        def out_start(j, slot):
            pltpu.async_copy(
                bufs[slot], out_hbm.at[pl.ds(row0 + j * CW, CW)], osems[slot])

        def out_wait(slot):
            pltpu.make_async_copy(
                bufs[slot], out_hbm.at[pl.ds(row0, CW)], osems[slot]).wait()

        pltpu.sync_copy(idx_hbm.at[wid], idx_v)

        for j in range(K):
            gather(j, j)

        for b in range(NB):
            j = b
            gather_wait(b)
            out_start(j, b)
            jj, tb = j + K, (j + K) % NB
            if jj >= NB:
                out_wait(tb)
            gather(jj, tb)

        @pl.loop(1, n_grp - 1)
        def _(g):
            for b in range(NB):
                j = g * NB + b
                gather_wait(b)
                out_start(j, b)
                tb = (b + K) % NB
                out_wait(tb)
                gather(j + K, tb)

        for b in range(NB):
            j = (n_grp - 1) * NB + b
            gather_wait(b)
            out_start(j, b)
            jj, tb = j + K, (j + K) % NB
            if jj < ch:
                out_wait(tb)
                gather(jj, tb)
        for b in range(NB):
            out_wait(b)

    return pl.kernel(
        body,
        out_type=jax.ShapeDtypeStruct((n_rows, dim), jnp.float32),
        mesh=plsc.VectorSubcoreMesh(core_axis_name="c", subcore_axis_name="s"),
        scratch_types=[
            pltpu.VMEM((ch, CW), jnp.int32),
            *[pltpu.VMEM((CW, dim), jnp.float32) for _ in range(NB)],
            *[pltpu.SemaphoreType.DMA for _ in range(2 * NB)],
        ],
        compiler_params=pltpu.CompilerParams(use_tc_tiling_on_sc=False),
    )


def kernel(batch, weight):
    batch_sz, hist = batch.shape
    vocab, dim = weight.shape
    n_rows = batch_sz * hist
    idx3d = batch.reshape(NW, n_rows // (NW * CW), CW)
    out = _build(n_rows, vocab, dim)(weight, idx3d)
    return out.reshape(batch_sz, hist, dim)
